# baseline (speedup 1.0000x reference)
import functools
import math

import jax
import jax.numpy as jnp
from jax import lax
from jax.experimental import pallas as pl
from jax.experimental.pallas import tpu as pltpu
from jax.experimental.pallas import tpu_sc as plsc

_SQRT_2_OVER_PI = math.sqrt(2.0 / math.pi)
_A = 0.044715
_B1 = -2.0 * _SQRT_2_OVER_PI
_B3 = -2.0 * _SQRT_2_OVER_PI * _A

_NC = 2
_NS = 16
_NW = _NC * _NS
_LANES = 16

_N_TOTAL = 2 * 2048 * 2048
_CHUNK = 16384
_UNROLL = 8


def _sc_gelu_body(x_hbm, o_hbm, in_v, out_v):
    wid = lax.axis_index("s") * _NC + lax.axis_index("c")
    per_w = _N_TOTAL // _NW
    base = wid * per_w
    n_chunks = per_w // _CHUNK

    def chunk_body(ci, carry):
        off = base + ci * _CHUNK
        pltpu.sync_copy(x_hbm.at[pl.ds(off, _CHUNK)], in_v)

        def inner(si, c2):
            s0 = si * (_LANES * _UNROLL)
            for k in range(_UNROLL):
                sl = pl.ds(s0 + k * _LANES, _LANES)
                x = in_v[sl]
                x2 = x * x
                x3 = x2 * x
                u = _B3 * x3 + _B1 * x
                y = x / (1.0 + jnp.exp(u))
                out_v[sl] = y
            return c2

        lax.fori_loop(0, _CHUNK // (_LANES * _UNROLL), inner, 0, unroll=1)
        pltpu.sync_copy(out_v, o_hbm.at[pl.ds(off, _CHUNK)])
        return carry

    lax.fori_loop(0, n_chunks, chunk_body, 0, unroll=1)


@functools.partial(jax.jit, static_argnums=())
def _sc_gelu(x_flat):
    mesh = plsc.VectorSubcoreMesh(core_axis_name="c", subcore_axis_name="s")
    f = functools.partial(
        pl.kernel,
        mesh=mesh,
        out_type=jax.ShapeDtypeStruct((_N_TOTAL,), jnp.float32),
        scratch_types=[
            pltpu.VMEM((_CHUNK,), jnp.float32),
            pltpu.VMEM((_CHUNK,), jnp.float32),
        ],
    )(_sc_gelu_body)
    return f(x_flat)


def kernel(x, log_k_ramp, log_g_high):
    del log_k_ramp, log_g_high
    y = _sc_gelu(x.reshape(-1))
    return y.reshape(x.shape)


# speedup vs baseline: 1.7704x; 1.7704x over previous
import functools
import math

import jax
import jax.numpy as jnp
from jax import lax
from jax.experimental import pallas as pl
from jax.experimental.pallas import tpu as pltpu
from jax.experimental.pallas import tpu_sc as plsc

_SQRT_2_OVER_PI = math.sqrt(2.0 / math.pi)
_A = 0.044715
_B1 = -2.0 * _SQRT_2_OVER_PI
_B3 = -2.0 * _SQRT_2_OVER_PI * _A

_NC = 2
_NS = 16
_NW = _NC * _NS
_LANES = 16

_ROWS = 4096
_COLS = 2048
_SC_ROWS = 512
_TC_ROWS = _ROWS - _SC_ROWS
_TC_BLOCK_ROWS = 512

_SC_N = _SC_ROWS * _COLS
_CHUNK = 16384
_UNROLL = 8


def _tc_gelu_block(x_ref, o_ref):
    x = x_ref[...]
    inner = _SQRT_2_OVER_PI * (x + 0.044715 * (x * x * x))
    o_ref[...] = 0.5 * x * (1.0 + jnp.tanh(inner))


def _tc_gelu(x2):
    rows, cols = x2.shape
    grid = (rows // _TC_BLOCK_ROWS,)
    return pl.pallas_call(
        _tc_gelu_block,
        grid=grid,
        in_specs=[pl.BlockSpec((_TC_BLOCK_ROWS, cols), lambda i: (i, 0))],
        out_specs=pl.BlockSpec((_TC_BLOCK_ROWS, cols), lambda i: (i, 0)),
        out_shape=jax.ShapeDtypeStruct((rows, cols), x2.dtype),
    )(x2)


def _sc_gelu_body(x_hbm, o_hbm, in_v, out_v):
    wid = lax.axis_index("s") * _NC + lax.axis_index("c")
    per_w = _SC_N // _NW
    base = wid * per_w
    n_chunks = per_w // _CHUNK

    def chunk_body(ci, carry):
        off = base + ci * _CHUNK
        pltpu.sync_copy(x_hbm.at[pl.ds(off, _CHUNK)], in_v)

        def inner(si, c2):
            s0 = si * (_LANES * _UNROLL)
            for k in range(_UNROLL):
                sl = pl.ds(s0 + k * _LANES, _LANES)
                x = in_v[sl]
                x2 = x * x
                u = x * (_B1 + _B3 * x2)
                y = x / (1.0 + jnp.exp(u))
                out_v[sl] = y
            return c2

        lax.fori_loop(0, _CHUNK // (_LANES * _UNROLL), inner, 0, unroll=1)
        pltpu.sync_copy(out_v, o_hbm.at[pl.ds(off, _CHUNK)])
        return carry

    lax.fori_loop(0, n_chunks, chunk_body, 0, unroll=1)


def _sc_gelu(x_flat):
    mesh = plsc.VectorSubcoreMesh(core_axis_name="c", subcore_axis_name="s")
    f = functools.partial(
        pl.kernel,
        mesh=mesh,
        out_type=jax.ShapeDtypeStruct((_SC_N,), jnp.float32),
        scratch_types=[
            pltpu.VMEM((_CHUNK,), jnp.float32),
            pltpu.VMEM((_CHUNK,), jnp.float32),
        ],
    )(_sc_gelu_body)
    return f(x_flat)


def kernel(x, log_k_ramp, log_g_high):
    del log_k_ramp, log_g_high
    x2 = x.reshape(_ROWS, _COLS)
    y_sc = _sc_gelu(x2[_TC_ROWS:].reshape(-1))
    y_tc = _tc_gelu(x2[:_TC_ROWS])
    y = jnp.concatenate([y_tc, y_sc.reshape(_SC_ROWS, _COLS)], axis=0)
    return y.reshape(x.shape)


# speedup vs baseline: 5.5858x; 3.1550x over previous
import math

import jax
import jax.numpy as jnp
from jax.experimental import pallas as pl

_SQRT_2_OVER_PI = math.sqrt(2.0 / math.pi)

_BLOCK_ROWS = 256


def _gelu_block_kernel(x_ref, o_ref):
    x = x_ref[...]
    inner = _SQRT_2_OVER_PI * (x + 0.044715 * (x * x * x))
    o_ref[...] = 0.5 * x * (1.0 + jnp.tanh(inner))


def kernel(x, log_k_ramp, log_g_high):
    del log_k_ramp, log_g_high
    orig_shape = x.shape
    x2 = x.reshape(-1, orig_shape[-1])
    rows, cols = x2.shape
    grid = (rows // _BLOCK_ROWS,)
    y2 = pl.pallas_call(
        _gelu_block_kernel,
        grid=grid,
        in_specs=[pl.BlockSpec((_BLOCK_ROWS, cols), lambda i: (i, 0))],
        out_specs=pl.BlockSpec((_BLOCK_ROWS, cols), lambda i: (i, 0)),
        out_shape=jax.ShapeDtypeStruct((rows, cols), x.dtype),
    )(x2)
    return y2.reshape(orig_shape)


# speedup vs baseline: 6.3899x; 1.1440x over previous
import math

import jax
import jax.numpy as jnp
from jax.experimental import pallas as pl

_SQRT_2_OVER_PI = math.sqrt(2.0 / math.pi)

_BLOCK_ROWS = 1024


def _gelu_block_kernel(x_ref, o_ref):
    x = x_ref[...]
    inner = _SQRT_2_OVER_PI * (x + 0.044715 * (x * x * x))
    o_ref[...] = 0.5 * x * (1.0 + jnp.tanh(inner))


def kernel(x, log_k_ramp, log_g_high):
    del log_k_ramp, log_g_high
    orig_shape = x.shape
    x2 = x.reshape(-1, orig_shape[-1])
    rows, cols = x2.shape
    grid = (rows // _BLOCK_ROWS,)
    y2 = pl.pallas_call(
        _gelu_block_kernel,
        grid=grid,
        in_specs=[pl.BlockSpec((_BLOCK_ROWS, cols), lambda i: (i, 0))],
        out_specs=pl.BlockSpec((_BLOCK_ROWS, cols), lambda i: (i, 0)),
        out_shape=jax.ShapeDtypeStruct((rows, cols), x.dtype),
    )(x2)
    return y2.reshape(orig_shape)
